# Initial kernel scaffold; baseline (speedup 1.0000x reference)
import jax
import jax.numpy as jnp
from jax.experimental import pallas as pl

N_NODES = 10000
IN_CH = 256
OUT_CH = 64
HEADS = 4
HC = HEADS * OUT_CH

_MM_ROWS = 400


def _proj_body(x_ref, w_ref, b_ref, o_ref):
    acc = jnp.dot(x_ref[...], w_ref[0, 0], preferred_element_type=jnp.float32)
    o_ref[0, 0] = acc + b_ref[0, 0]


def _project(x, W_all, b_all):
    n_blk = N_NODES // _MM_ROWS
    return pl.pallas_call(
        _proj_body,
        grid=(2, 2, n_blk),
        in_specs=[
            pl.BlockSpec((_MM_ROWS, IN_CH), lambda m, h, i: (i, 0)),
            pl.BlockSpec((1, 1, IN_CH, 128), lambda m, h, i: (m, h, 0, 0)),
            pl.BlockSpec((1, 1, 128), lambda m, h, i: (m, h, 0)),
        ],
        out_specs=pl.BlockSpec((1, 1, _MM_ROWS, 128), lambda m, h, i: (m, h, i, 0)),
        out_shape=jax.ShapeDtypeStruct((2, 2, N_NODES, 128), jnp.float32),
    )(x, W_all, b_all)


def _ln_elu_body(h_ref, bias_ref, gamma_ref, beta_ref, o_ref):
    full = jnp.concatenate([h_ref[0], h_ref[1]], axis=-1) + bias_ref[...]
    mean = jnp.mean(full, axis=-1, keepdims=True)
    var = jnp.mean((full - mean) ** 2, axis=-1, keepdims=True)
    y = (full - mean) / jnp.sqrt(var + 1e-5) * gamma_ref[...] + beta_ref[...]
    o_ref[...] = jnp.where(y > 0, y, jnp.expm1(y))


def _ln_elu(halves, bias, gamma, beta):
    n_blk = N_NODES // _MM_ROWS
    return pl.pallas_call(
        _ln_elu_body,
        grid=(n_blk,),
        in_specs=[
            pl.BlockSpec((2, _MM_ROWS, 128), lambda i: (0, i, 0)),
            pl.BlockSpec((HC,), lambda i: (0,)),
            pl.BlockSpec((HC,), lambda i: (0,)),
            pl.BlockSpec((HC,), lambda i: (0,)),
        ],
        out_specs=pl.BlockSpec((_MM_ROWS, HC), lambda i: (i, 0)),
        out_shape=jax.ShapeDtypeStruct((N_NODES, HC), jnp.float32),
    )(halves, bias, gamma, beta)


def kernel(x, edge_index, W_l, b_l, W_r, b_r, att, bias, gamma, beta):
    N = x.shape[0]
    H, C = att.shape[1], att.shape[2]

    W_all = jnp.stack([
        W_l.reshape(IN_CH, 2, 128).transpose(1, 0, 2),
        W_r.reshape(IN_CH, 2, 128).transpose(1, 0, 2),
    ])
    b_all = jnp.stack([b_l.reshape(2, 128), b_r.reshape(2, 128)])

    proj = _project(x, W_all, b_all)
    x_l = jnp.concatenate([proj[0, 0], proj[0, 1]], axis=-1).reshape(N, H, C)
    x_r = jnp.concatenate([proj[1, 0], proj[1, 1]], axis=-1).reshape(N, H, C)

    loops = jnp.arange(N, dtype=edge_index.dtype)
    src = jnp.concatenate([edge_index[0], loops]).astype(jnp.int32)
    dst = jnp.concatenate([edge_index[1], loops]).astype(jnp.int32)

    e = x_l[src] + x_r[dst]
    e = jax.nn.leaky_relu(e, negative_slope=0.2)
    alpha = jnp.sum(e * att[0], axis=-1)
    w = jnp.exp(alpha)
    asum = jax.ops.segment_sum(w, dst, num_segments=N)
    num = jax.ops.segment_sum(w[:, :, None] * x_l[src], dst, num_segments=N)
    out = num / (asum[:, :, None] + 1e-16)

    halves = out.reshape(N, HC).reshape(N, 2, 128).transpose(1, 0, 2)
    return _ln_elu(halves, bias, gamma, beta)


# speedup vs baseline: 1.1135x; 1.1135x over previous
import jax
import jax.numpy as jnp
from jax.experimental import pallas as pl

N_NODES = 10000
HC = 256
_ROWS = 400


def _ln_elu_body(h_ref, bias_ref, gamma_ref, beta_ref, o_ref):
    full = h_ref[...] + bias_ref[0]
    mean = jnp.mean(full, axis=-1, keepdims=True)
    var = jnp.mean((full - mean) ** 2, axis=-1, keepdims=True)
    y = (full - mean) / jnp.sqrt(var + 1e-5) * gamma_ref[0] + beta_ref[0]
    o_ref[...] = jnp.where(y > 0, y, jnp.exp(jnp.minimum(y, 0.0)) - 1.0)


def _ln_elu(h, bias, gamma, beta):
    n_blk = N_NODES // _ROWS
    return pl.pallas_call(
        _ln_elu_body,
        grid=(n_blk,),
        in_specs=[
            pl.BlockSpec((_ROWS, HC), lambda i: (i, 0)),
            pl.BlockSpec((1, HC), lambda i: (0, 0)),
            pl.BlockSpec((1, HC), lambda i: (0, 0)),
            pl.BlockSpec((1, HC), lambda i: (0, 0)),
        ],
        out_specs=pl.BlockSpec((_ROWS, HC), lambda i: (i, 0)),
        out_shape=jax.ShapeDtypeStruct((N_NODES, HC), jnp.float32),
    )(h, bias.reshape(1, HC), gamma.reshape(1, HC), beta.reshape(1, HC))


def _mm_body(x_ref, w_ref, b_ref, o_ref):
    o_ref[...] = jnp.dot(x_ref[...], w_ref[...],
                         preferred_element_type=jnp.float32) + b_ref[0]


def _mm(x, W, b):
    n_blk = N_NODES // _ROWS
    return pl.pallas_call(
        _mm_body,
        grid=(n_blk,),
        in_specs=[
            pl.BlockSpec((_ROWS, 256), lambda i: (i, 0)),
            pl.BlockSpec((256, 256), lambda i: (0, 0)),
            pl.BlockSpec((1, 256), lambda i: (0, 0)),
        ],
        out_specs=pl.BlockSpec((_ROWS, 256), lambda i: (i, 0)),
        out_shape=jax.ShapeDtypeStruct((N_NODES, 256), jnp.float32),
    )(x, W, b.reshape(1, 256))


def kernel(x, edge_index, W_l, b_l, W_r, b_r, att, bias, gamma, beta):
    N = x.shape[0]
    H, C = att.shape[1], att.shape[2]
    x_l = _mm(x, W_l, b_l).reshape(N, H, C)
    x_r = _mm(x, W_r, b_r).reshape(N, H, C)
    loops = jnp.arange(N, dtype=edge_index.dtype)
    src = jnp.concatenate([edge_index[0], loops])
    dst = jnp.concatenate([edge_index[1], loops])
    e = x_l[src] + x_r[dst]
    e = jax.nn.leaky_relu(e, negative_slope=0.2)
    alpha = jnp.sum(e * att[0], axis=-1)
    w = jnp.exp(alpha)
    asum = jax.ops.segment_sum(w, dst, num_segments=N)
    num = jax.ops.segment_sum(w[:, :, None] * x_l[src], dst, num_segments=N)
    out = (num / (asum[:, :, None] + 1e-16)).reshape(N, H * C)
    return _ln_elu(out, bias, gamma, beta)


# speedup vs baseline: 7.4065x; 6.6517x over previous
import dataclasses
import functools

import jax
import jax.numpy as jnp
from jax import lax
from jax.experimental import pallas as pl
from jax.experimental.pallas import tpu as pltpu
from jax.experimental.pallas import tpu_sc as plsc

N_NODES = 10000
IN_CH = 256
HC = 256
HALF = 128
E_RAW = 160000
E_TOT = E_RAW + N_NODES

N_TILES = 16
CHUNK = 64
CH_PER_TILE = 168
E_PAD = N_TILES * CH_PER_TILE * CHUNK

PH_NODES = N_NODES // 2
ACC_N_ROWS = 5248
DUMMY_N = 5120
ACC_W_ROWS = 640
DUMMY_W = 630
NZCH = ACC_N_ROWS // CHUNK
NZW = ACC_W_ROWS // CHUNK
OCH = 40
NOCH = PH_NODES // OCH
WCP = 10

_ROWS = 400


def _mm_body(x_ref, w_ref, b_ref, o_ref):
    o_ref[...] = jnp.dot(x_ref[...], w_ref[...],
                         preferred_element_type=jnp.float32) + b_ref[0]


def _mm(x, W, b):
    n_blk = N_NODES // _ROWS
    return pl.pallas_call(
        _mm_body,
        grid=(n_blk,),
        in_specs=[
            pl.BlockSpec((_ROWS, IN_CH), lambda i: (i, 0)),
            pl.BlockSpec((IN_CH, HC), lambda i: (0, 0)),
            pl.BlockSpec((1, HC), lambda i: (0, 0)),
        ],
        out_specs=pl.BlockSpec((_ROWS, HC), lambda i: (i, 0)),
        out_shape=jax.ShapeDtypeStruct((N_NODES, HC), jnp.float32),
    )(x, W, b.reshape(1, HC))


def _ln_elu_body(h_ref, w_ref, bias_ref, gamma_ref, beta_ref, o_ref):
    num = jnp.concatenate([h_ref[0], h_ref[1]], axis=-1)
    pieces = []
    for h in range(4):
        inv = 1.0 / (w_ref[:, h:h + 1] + 1e-16)
        pieces.append(num[:, 64 * h:64 * (h + 1)] * inv)
    full = jnp.concatenate(pieces, axis=-1) + bias_ref[0]
    mean = jnp.mean(full, axis=-1, keepdims=True)
    var = jnp.mean((full - mean) ** 2, axis=-1, keepdims=True)
    y = (full - mean) / jnp.sqrt(var + 1e-5) * gamma_ref[0] + beta_ref[0]
    o_ref[...] = jnp.where(y > 0, y, jnp.exp(jnp.minimum(y, 0.0)) - 1.0)


def _ln_elu(halves, w4, bias, gamma, beta):
    n_blk = N_NODES // _ROWS
    return pl.pallas_call(
        _ln_elu_body,
        grid=(n_blk,),
        in_specs=[
            pl.BlockSpec((2, _ROWS, HALF), lambda i: (0, i, 0)),
            pl.BlockSpec((_ROWS, 4), lambda i: (i, 0)),
            pl.BlockSpec((1, HC), lambda i: (0, 0)),
            pl.BlockSpec((1, HC), lambda i: (0, 0)),
            pl.BlockSpec((1, HC), lambda i: (0, 0)),
        ],
        out_specs=pl.BlockSpec((_ROWS, HC), lambda i: (i, 0)),
        out_shape=jax.ShapeDtypeStruct((N_NODES, HC), jnp.float32),
    )(halves, w4, bias.reshape(1, HC), gamma.reshape(1, HC),
      beta.reshape(1, HC))


def _sc_edge_body(xl_hbm, xr_hbm, src_hbm, dst_hbm, oh0_hbm, oh1_hbm,
                  att_hbm, outn_hbm, outw_hbm,
                  srcraw, dstraw, srcidx, dgidx, dsbuf, dswbuf,
                  oh0buf, oh1buf, attbuf, xlrows, xrrows, stg, stgw,
                  accn, accw):
    c = lax.axis_index("c")
    s = lax.axis_index("s")

    pltpu.sync_copy(att_hbm.at[pl.ds(c * HALF, HALF)], attbuf)

    for p in range(2):
        @pl.loop(0, CHUNK)
        def _zs(i):
            for j in range(8):
                stg[i, pl.ds(16 * j, 16)] = jnp.zeros((16,), jnp.float32)
                stgw[i, pl.ds(16 * j, 16)] = jnp.zeros((16,), jnp.float32)

        for k in range(6):
            zid = s + N_TILES * k

            @pl.when(zid < NZCH)
            def _zn():
                pltpu.sync_copy(stg, accn.at[pl.ds(zid * CHUNK, CHUNK)])

        @pl.when(s < NZW)
        def _zw():
            pltpu.sync_copy(stgw, accw.at[pl.ds(s * CHUNK, CHUNK)])

        plsc.subcore_barrier()

        @pl.loop(0, CH_PER_TILE)
        def _edge_chunk(g):
            base = (s * CH_PER_TILE + g) * CHUNK
            pltpu.sync_copy(src_hbm.at[pl.ds(base, CHUNK)], srcraw)
            pltpu.sync_copy(dst_hbm.at[pl.ds(base, CHUNK)], dstraw)
            pltpu.sync_copy(oh0_hbm.at[pl.ds(base, CHUNK)], oh0buf)
            pltpu.sync_copy(oh1_hbm.at[pl.ds(base, CHUNK)], oh1buf)

            coff = c * N_NODES

            @pl.loop(0, 8)
            def _ids(t):
                sl = pl.ds(16 * t, 16)
                dv = dstraw[sl]
                srcidx[sl] = srcraw[sl] + coff
                dgidx[sl] = jnp.minimum(dv, N_NODES - 1) + coff
                lv = dv - p * PH_NODES
                ok = (lv >= 0) & (lv < PH_NODES)
                spread = dv & 63
                dsbuf[sl] = jnp.where(ok, lv, DUMMY_N + spread)
                dswbuf[sl] = jnp.where(ok, lv >> 3, DUMMY_W + (dv & 7))

            pltpu.sync_copy(xl_hbm.at[srcidx], xlrows)
            pltpu.sync_copy(xr_hbm.at[dgidx], xrrows)

            @pl.loop(0, CHUNK)
            def _edge(e):
                xs = [xlrows[e, pl.ds(16 * j, 16)] for j in range(8)]
                ws = []
                for q in range(2):
                    tsum = None
                    for j in range(4 * q, 4 * q + 4):
                        z = xs[j] + xrrows[e, pl.ds(16 * j, 16)]
                        t = jnp.maximum(z, z * 0.2) * attbuf[pl.ds(16 * j, 16)]
                        tsum = t if tsum is None else tsum + t
                    a = jnp.sum(tsum)
                    ws.append(jnp.exp(jnp.full((16,), a, jnp.float32)))
                for j in range(8):
                    stg[e, pl.ds(16 * j, 16)] = xs[j] * ws[j // 4]
                stgw[e, pl.ds(0, 16)] = (ws[0] * oh0buf[e, pl.ds(0, 16)] +
                                         ws[1] * oh1buf[e, pl.ds(0, 16)])

            pltpu.sync_copy(stg, accn.at[dsbuf], add=True)
            pltpu.sync_copy(stgw, accw.at[dswbuf], add=True)

        plsc.subcore_barrier()

        for k in range(8):
            oid = s + N_TILES * k

            @pl.when(oid < NOCH)
            def _on():
                pltpu.sync_copy(
                    accn.at[pl.ds(oid * OCH, OCH)],
                    outn_hbm.at[pl.ds(c * N_NODES + p * PH_NODES + oid * OCH,
                                      OCH)])

        @pl.when(s < WCP)
        def _ow():
            pltpu.sync_copy(
                accw.at[pl.ds(s * CHUNK, CHUNK)],
                outw_hbm.at[pl.ds(((2 * c + p) * WCP + s) * CHUNK, CHUNK)])

        plsc.subcore_barrier()


_sc_cp = pltpu.CompilerParams()
if "needs_layout_passes" in pltpu.CompilerParams.__dataclass_fields__:
    _sc_cp = dataclasses.replace(_sc_cp, needs_layout_passes=False)

_sc_edge = functools.partial(
    pl.kernel,
    compiler_params=_sc_cp,
    out_type=(
        jax.ShapeDtypeStruct((2 * N_NODES, HALF), jnp.float32),
        jax.ShapeDtypeStruct((4 * WCP * CHUNK, HALF), jnp.float32),
    ),
    mesh=plsc.VectorSubcoreMesh(core_axis_name="c", subcore_axis_name="s"),
    scratch_types=[
        pltpu.VMEM((CHUNK,), jnp.int32),
        pltpu.VMEM((CHUNK,), jnp.int32),
        pltpu.VMEM((CHUNK,), jnp.int32),
        pltpu.VMEM((CHUNK,), jnp.int32),
        pltpu.VMEM((CHUNK,), jnp.int32),
        pltpu.VMEM((CHUNK,), jnp.int32),
        pltpu.VMEM((CHUNK, 16), jnp.float32),
        pltpu.VMEM((CHUNK, 16), jnp.float32),
        pltpu.VMEM((HALF,), jnp.float32),
        pltpu.VMEM((CHUNK, HALF), jnp.float32),
        pltpu.VMEM((CHUNK, HALF), jnp.float32),
        pltpu.VMEM((CHUNK, HALF), jnp.float32),
        pltpu.VMEM((CHUNK, HALF), jnp.float32),
        pltpu.VMEM_SHARED((ACC_N_ROWS, HALF), jnp.float32),
        pltpu.VMEM_SHARED((ACC_W_ROWS, HALF), jnp.float32),
    ],
)(_sc_edge_body)


def kernel(x, edge_index, W_l, b_l, W_r, b_r, att, bias, gamma, beta):
    ei = edge_index.astype(jnp.int32)
    loops = jnp.arange(N_NODES, dtype=jnp.int32)
    src = jnp.concatenate([ei[0], loops])
    dst = jnp.concatenate([ei[1], loops])
    pad = E_PAD - E_TOT
    srcp = jnp.concatenate([src, jnp.zeros((pad,), jnp.int32)])
    dstp = jnp.concatenate([dst, jnp.full((pad,), 2 * N_NODES, jnp.int32)])
    lane0 = 2 * (dstp & 7)
    lanes = jnp.arange(16, dtype=jnp.int32)
    oh0 = (lanes[None, :] == lane0[:, None]).astype(jnp.float32)
    oh1 = (lanes[None, :] == (lane0 + 1)[:, None]).astype(jnp.float32)
    att1 = att.reshape(HC)

    xl = _mm(x, W_l, b_l)
    xr = _mm(x, W_r, b_r)
    xlt = xl.reshape(N_NODES, 2, HALF).transpose(1, 0, 2).reshape(-1, HALF)
    xrt = xr.reshape(N_NODES, 2, HALF).transpose(1, 0, 2).reshape(-1, HALF)

    outn, outw = _sc_edge(xlt, xrt, srcp, dstp, oh0, oh1, att1)
    halves = outn.reshape(2, N_NODES, HALF)
    wq = outw.reshape(2, 2, WCP * CHUNK, HALF)[:, :, :, :16]
    wq = wq.reshape(2, 2, WCP * CHUNK * 8, 2)[:, :, :PH_NODES, :]
    wq = wq.reshape(2, N_NODES, 2)
    w4 = jnp.concatenate([wq[0], wq[1]], axis=-1)

    return _ln_elu(halves, w4, bias, gamma, beta)


# speedup vs baseline: 12.7547x; 1.7221x over previous
import dataclasses
import functools

import jax
import jax.numpy as jnp
from jax import lax
from jax.experimental import pallas as pl
from jax.experimental.pallas import tpu as pltpu
from jax.experimental.pallas import tpu_sc as plsc

N_NODES = 10000
IN_CH = 256
HC = 256
HALF = 128
E_RAW = 160000
E_TOT = E_RAW + N_NODES

N_TILES = 16
CHUNK = 64
CH_PER_TILE = 168
E_PAD = N_TILES * CH_PER_TILE * CHUNK

PH_NODES = N_NODES // 2
ACC_N_ROWS = 5248
DUMMY_N = 5120
ACC_W_ROWS = 640
DUMMY_W = 630
NZCH = ACC_N_ROWS // CHUNK
NZW = ACC_W_ROWS // CHUNK
OCH = 40
NOCH = PH_NODES // OCH
WCP = 10

_ROWS = 400


def _mm_body(x_ref, w_ref, b_ref, o_ref):
    o_ref[...] = jnp.dot(x_ref[...], w_ref[...],
                         preferred_element_type=jnp.float32) + b_ref[0]


def _mm(x, W, b):
    n_blk = N_NODES // _ROWS
    return pl.pallas_call(
        _mm_body,
        grid=(n_blk,),
        in_specs=[
            pl.BlockSpec((_ROWS, IN_CH), lambda i: (i, 0)),
            pl.BlockSpec((IN_CH, HC), lambda i: (0, 0)),
            pl.BlockSpec((1, HC), lambda i: (0, 0)),
        ],
        out_specs=pl.BlockSpec((_ROWS, HC), lambda i: (i, 0)),
        out_shape=jax.ShapeDtypeStruct((N_NODES, HC), jnp.float32),
    )(x, W, b.reshape(1, HC))


def _ln_elu_body(h_ref, w_ref, bias_ref, gamma_ref, beta_ref, o_ref):
    num = jnp.concatenate([h_ref[0], h_ref[1]], axis=-1)
    pieces = []
    for h in range(4):
        inv = 1.0 / (w_ref[:, h:h + 1] + 1e-16)
        pieces.append(num[:, 64 * h:64 * (h + 1)] * inv)
    full = jnp.concatenate(pieces, axis=-1) + bias_ref[0]
    mean = jnp.mean(full, axis=-1, keepdims=True)
    var = jnp.mean((full - mean) ** 2, axis=-1, keepdims=True)
    y = (full - mean) / jnp.sqrt(var + 1e-5) * gamma_ref[0] + beta_ref[0]
    o_ref[...] = jnp.where(y > 0, y, jnp.exp(jnp.minimum(y, 0.0)) - 1.0)


def _ln_elu(halves, w4, bias, gamma, beta):
    n_blk = N_NODES // _ROWS
    return pl.pallas_call(
        _ln_elu_body,
        grid=(n_blk,),
        in_specs=[
            pl.BlockSpec((2, _ROWS, HALF), lambda i: (0, i, 0)),
            pl.BlockSpec((_ROWS, 4), lambda i: (i, 0)),
            pl.BlockSpec((1, HC), lambda i: (0, 0)),
            pl.BlockSpec((1, HC), lambda i: (0, 0)),
            pl.BlockSpec((1, HC), lambda i: (0, 0)),
        ],
        out_specs=pl.BlockSpec((_ROWS, HC), lambda i: (i, 0)),
        out_shape=jax.ShapeDtypeStruct((N_NODES, HC), jnp.float32),
    )(halves, w4, bias.reshape(1, HC), gamma.reshape(1, HC),
      beta.reshape(1, HC))


def _sc_edge_body(xl_hbm, xr_hbm, src_hbm, dst_hbm, oh0_hbm, oh1_hbm,
                  att_hbm, outn_hbm, outw_hbm,
                  srcraw, dstraw, srcidx, dgidx, dsbuf, dswbuf,
                  oh0buf, oh1buf, attbuf, xlrows, xrrows, stg, stgw,
                  accn, accw, sem_i0, sem_i1, sem_i2, sem_i3,
                  sem_g0, sem_g1, sem_s0, sem_s1):
    c = lax.axis_index("c")
    s = lax.axis_index("s")

    pltpu.sync_copy(att_hbm.at[pl.ds(c * HALF, HALF)], attbuf)

    for p in range(2):
        @pl.loop(0, CHUNK)
        def _zs(i):
            for j in range(8):
                stg[i, pl.ds(16 * j, 16)] = jnp.zeros((16,), jnp.float32)
                stgw[i, pl.ds(16 * j, 16)] = jnp.zeros((16,), jnp.float32)

        for k in range(6):
            zid = s + N_TILES * k

            @pl.when(zid < NZCH)
            def _zn():
                pltpu.sync_copy(stg, accn.at[pl.ds(zid * CHUNK, CHUNK)])

        @pl.when(s < NZW)
        def _zw():
            pltpu.sync_copy(stgw, accw.at[pl.ds(s * CHUNK, CHUNK)])

        plsc.subcore_barrier()

        @pl.loop(0, CH_PER_TILE)
        def _edge_chunk(g):
            base = (s * CH_PER_TILE + g) * CHUNK
            hi0 = pltpu.async_copy(src_hbm.at[pl.ds(base, CHUNK)], srcraw,
                                   sem_i0)
            hi1 = pltpu.async_copy(dst_hbm.at[pl.ds(base, CHUNK)], dstraw,
                                   sem_i1)
            hi2 = pltpu.async_copy(oh0_hbm.at[pl.ds(base, CHUNK)], oh0buf,
                                   sem_i2)
            hi3 = pltpu.async_copy(oh1_hbm.at[pl.ds(base, CHUNK)], oh1buf,
                                   sem_i3)
            hi0.wait()
            hi1.wait()

            coff = c * N_NODES

            @pl.loop(0, CHUNK // 16)
            def _ids(t):
                sl = pl.ds(16 * t, 16)
                dv = dstraw[sl]
                srcidx[sl] = srcraw[sl] + coff
                dgidx[sl] = jnp.minimum(dv, N_NODES - 1) + coff
                lv = dv - p * PH_NODES
                ok = (lv >= 0) & (lv < PH_NODES)
                spread = dv & 63
                dsbuf[sl] = jnp.where(ok, lv, DUMMY_N + spread)
                dswbuf[sl] = jnp.where(ok, lv >> 3, DUMMY_W + (dv & 7))

            hg0 = pltpu.async_copy(xl_hbm.at[srcidx], xlrows, sem_g0)
            hg1 = pltpu.async_copy(xr_hbm.at[dgidx], xrrows, sem_g1)

            @pl.when(g > 0)
            def _drain():
                pltpu.make_async_copy(
                    xl_hbm.at[pl.ds(0, CHUNK)], stg, sem_s0).wait()
                pltpu.make_async_copy(
                    oh0_hbm.at[pl.ds(0, CHUNK)], stgw, sem_s1).wait()

            hi2.wait()
            hi3.wait()
            hg0.wait()
            hg1.wait()

            @pl.loop(0, CHUNK)
            def _edge(e):
                xs = [xlrows[e, pl.ds(16 * j, 16)] for j in range(8)]
                ws = []
                for q in range(2):
                    tsum = None
                    for j in range(4 * q, 4 * q + 4):
                        z = xs[j] + xrrows[e, pl.ds(16 * j, 16)]
                        t = jnp.maximum(z, z * 0.2) * attbuf[pl.ds(16 * j, 16)]
                        tsum = t if tsum is None else tsum + t
                    a = jnp.sum(tsum)
                    ws.append(jnp.exp(jnp.full((16,), a, jnp.float32)))
                for j in range(8):
                    stg[e, pl.ds(16 * j, 16)] = xs[j] * ws[j // 4]
                stgw[e, pl.ds(0, 16)] = (ws[0] * oh0buf[e, pl.ds(0, 16)] +
                                         ws[1] * oh1buf[e, pl.ds(0, 16)])

            pltpu.async_copy(stg, accn.at[dsbuf], sem_s0, add=True)
            pltpu.async_copy(stgw, accw.at[dswbuf], sem_s1, add=True)

        pltpu.make_async_copy(xl_hbm.at[pl.ds(0, CHUNK)], stg, sem_s0).wait()
        pltpu.make_async_copy(oh0_hbm.at[pl.ds(0, CHUNK)], stgw,
                              sem_s1).wait()

        plsc.subcore_barrier()

        for k in range(8):
            oid = s + N_TILES * k

            @pl.when(oid < NOCH)
            def _on():
                pltpu.sync_copy(
                    accn.at[pl.ds(oid * OCH, OCH)],
                    outn_hbm.at[pl.ds(c * N_NODES + p * PH_NODES + oid * OCH,
                                      OCH)])

        @pl.when(s < WCP)
        def _ow():
            pltpu.sync_copy(
                accw.at[pl.ds(s * CHUNK, CHUNK)],
                outw_hbm.at[pl.ds(((2 * c + p) * WCP + s) * CHUNK, CHUNK)])

        plsc.subcore_barrier()


_sc_cp = pltpu.CompilerParams()
if "needs_layout_passes" in pltpu.CompilerParams.__dataclass_fields__:
    _sc_cp = dataclasses.replace(_sc_cp, needs_layout_passes=False)

_sc_edge = functools.partial(
    pl.kernel,
    compiler_params=_sc_cp,
    out_type=(
        jax.ShapeDtypeStruct((2 * N_NODES, HALF), jnp.float32),
        jax.ShapeDtypeStruct((4 * WCP * CHUNK, HALF), jnp.float32),
    ),
    mesh=plsc.VectorSubcoreMesh(core_axis_name="c", subcore_axis_name="s"),
    scratch_types=[
        pltpu.VMEM((CHUNK,), jnp.int32),
        pltpu.VMEM((CHUNK,), jnp.int32),
        pltpu.VMEM((CHUNK,), jnp.int32),
        pltpu.VMEM((CHUNK,), jnp.int32),
        pltpu.VMEM((CHUNK,), jnp.int32),
        pltpu.VMEM((CHUNK,), jnp.int32),
        pltpu.VMEM((CHUNK, 16), jnp.float32),
        pltpu.VMEM((CHUNK, 16), jnp.float32),
        pltpu.VMEM((HALF,), jnp.float32),
        pltpu.VMEM((CHUNK, HALF), jnp.float32),
        pltpu.VMEM((CHUNK, HALF), jnp.float32),
        pltpu.VMEM((CHUNK, HALF), jnp.float32),
        pltpu.VMEM((CHUNK, HALF), jnp.float32),
        pltpu.VMEM_SHARED((ACC_N_ROWS, HALF), jnp.float32),
        pltpu.VMEM_SHARED((ACC_W_ROWS, HALF), jnp.float32),
        pltpu.SemaphoreType.DMA,
        pltpu.SemaphoreType.DMA,
        pltpu.SemaphoreType.DMA,
        pltpu.SemaphoreType.DMA,
        pltpu.SemaphoreType.DMA,
        pltpu.SemaphoreType.DMA,
        pltpu.SemaphoreType.DMA,
        pltpu.SemaphoreType.DMA,
    ],
)(_sc_edge_body)


def kernel(x, edge_index, W_l, b_l, W_r, b_r, att, bias, gamma, beta):
    ei = edge_index.astype(jnp.int32)
    loops = jnp.arange(N_NODES, dtype=jnp.int32)
    src = jnp.concatenate([ei[0], loops])
    dst = jnp.concatenate([ei[1], loops])
    pad = E_PAD - E_TOT
    srcp = jnp.concatenate([src, jnp.zeros((pad,), jnp.int32)])
    dstp = jnp.concatenate([dst, jnp.full((pad,), 2 * N_NODES, jnp.int32)])
    lane0 = 2 * (dstp & 7)
    lanes = jnp.arange(16, dtype=jnp.int32)
    oh0 = (lanes[None, :] == lane0[:, None]).astype(jnp.float32)
    oh1 = (lanes[None, :] == (lane0 + 1)[:, None]).astype(jnp.float32)
    att1 = att.reshape(HC)

    xl = _mm(x, W_l, b_l)
    xr = _mm(x, W_r, b_r)
    xlt = xl.reshape(N_NODES, 2, HALF).transpose(1, 0, 2).reshape(-1, HALF)
    xrt = xr.reshape(N_NODES, 2, HALF).transpose(1, 0, 2).reshape(-1, HALF)

    outn, outw = _sc_edge(xlt, xrt, srcp, dstp, oh0, oh1, att1)
    halves = outn.reshape(2, N_NODES, HALF)
    wq = outw.reshape(2, 2, WCP * CHUNK, HALF)[:, :, :, :16]
    wq = wq.reshape(2, 2, WCP * CHUNK * 8, 2)[:, :, :PH_NODES, :]
    wq = wq.reshape(2, N_NODES, 2)
    w4 = jnp.concatenate([wq[0], wq[1]], axis=-1)

    return _ln_elu(halves, w4, bias, gamma, beta)


# speedup vs baseline: 13.5110x; 1.0593x over previous
import dataclasses
import functools

import jax
import jax.numpy as jnp
from jax import lax
from jax.experimental import pallas as pl
from jax.experimental.pallas import tpu as pltpu
from jax.experimental.pallas import tpu_sc as plsc

N_NODES = 10000
IN_CH = 256
HC = 256
HALF = 128
E_RAW = 160000
E_TOT = E_RAW + N_NODES

N_TILES = 16
CHUNK = 64
CH_PER_TILE = 168
NB2 = CH_PER_TILE // 2
E_PAD = N_TILES * CH_PER_TILE * CHUNK

PH_NODES = N_NODES // 2
ACC_N_ROWS = 5184
DUMMY_N = 5120
ACC_W_ROWS = 640
DUMMY_W = 630
NZCH = ACC_N_ROWS // CHUNK
NZW = ACC_W_ROWS // CHUNK
OCH = 40
NOCH = PH_NODES // OCH
WCP = ACC_W_ROWS // CHUNK

_ROWS = 400


def _mm_body(x_ref, w_ref, b_ref, o_ref):
    o_ref[...] = jnp.dot(x_ref[...], w_ref[...],
                         preferred_element_type=jnp.float32) + b_ref[0]


def _mm(x, W, b):
    n_blk = N_NODES // _ROWS
    return pl.pallas_call(
        _mm_body,
        grid=(n_blk,),
        in_specs=[
            pl.BlockSpec((_ROWS, IN_CH), lambda i: (i, 0)),
            pl.BlockSpec((IN_CH, HC), lambda i: (0, 0)),
            pl.BlockSpec((1, HC), lambda i: (0, 0)),
        ],
        out_specs=pl.BlockSpec((_ROWS, HC), lambda i: (i, 0)),
        out_shape=jax.ShapeDtypeStruct((N_NODES, HC), jnp.float32),
    )(x, W, b.reshape(1, HC))


def _ln_elu_body(h_ref, w_ref, bias_ref, gamma_ref, beta_ref, o_ref):
    num = jnp.concatenate([h_ref[0], h_ref[1]], axis=-1)
    pieces = []
    for h in range(4):
        inv = 1.0 / (w_ref[:, h:h + 1] + 1e-16)
        pieces.append(num[:, 64 * h:64 * (h + 1)] * inv)
    full = jnp.concatenate(pieces, axis=-1) + bias_ref[0]
    mean = jnp.mean(full, axis=-1, keepdims=True)
    var = jnp.mean((full - mean) ** 2, axis=-1, keepdims=True)
    y = (full - mean) / jnp.sqrt(var + 1e-5) * gamma_ref[0] + beta_ref[0]
    o_ref[...] = jnp.where(y > 0, y, jnp.exp(jnp.minimum(y, 0.0)) - 1.0)


def _ln_elu(halves, w4, bias, gamma, beta):
    n_blk = N_NODES // _ROWS
    return pl.pallas_call(
        _ln_elu_body,
        grid=(n_blk,),
        in_specs=[
            pl.BlockSpec((2, _ROWS, HALF), lambda i: (0, i, 0)),
            pl.BlockSpec((_ROWS, 4), lambda i: (i, 0)),
            pl.BlockSpec((1, HC), lambda i: (0, 0)),
            pl.BlockSpec((1, HC), lambda i: (0, 0)),
            pl.BlockSpec((1, HC), lambda i: (0, 0)),
        ],
        out_specs=pl.BlockSpec((_ROWS, HC), lambda i: (i, 0)),
        out_shape=jax.ShapeDtypeStruct((N_NODES, HC), jnp.float32),
    )(halves, w4, bias.reshape(1, HC), gamma.reshape(1, HC),
      beta.reshape(1, HC))


def _sc_edge_body(xl_hbm, xr_hbm, src_hbm, dst_hbm, ohc_hbm, att_hbm,
                  outn_hbm, outw_hbm,
                  srcrawA, dstrawA, ohbufA, srcidxA, dgidxA, dsbufA, dswbufA,
                  xlrowsA, xrrowsA,
                  srcrawB, dstrawB, ohbufB, srcidxB, dgidxB, dsbufB, dswbufB,
                  xlrowsB, xrrowsB,
                  attbuf, stg, stgw, accn, accw,
                  sem_iA, sem_iB, sem_gA, sem_gB, sem_s0, sem_s1):
    c = lax.axis_index("c")
    s = lax.axis_index("s")

    A = (srcrawA, dstrawA, ohbufA, srcidxA, dgidxA, dsbufA, dswbufA,
         xlrowsA, xrrowsA, sem_iA, sem_gA)
    B = (srcrawB, dstrawB, ohbufB, srcidxB, dgidxB, dsbufB, dswbufB,
         xlrowsB, xrrowsB, sem_iB, sem_gB)

    def fetch(S, chunk):
        srcraw, dstraw, ohbuf, _, _, _, _, _, _, sem_i, _ = S
        base = (s * CH_PER_TILE + chunk) * CHUNK
        pltpu.async_copy(src_hbm.at[pl.ds(base, CHUNK)], srcraw, sem_i)
        pltpu.async_copy(dst_hbm.at[pl.ds(base, CHUNK)], dstraw, sem_i)
        pltpu.async_copy(ohc_hbm.at[pl.ds(base, CHUNK)], ohbuf, sem_i)

    def stage(S, p):
        (srcraw, dstraw, ohbuf, srcidx, dgidx, dsbuf, dswbuf,
         xlrows, xrrows, sem_i, sem_g) = S
        pltpu.make_async_copy(src_hbm.at[pl.ds(0, CHUNK)], srcraw,
                              sem_i).wait()
        pltpu.make_async_copy(dst_hbm.at[pl.ds(0, CHUNK)], dstraw,
                              sem_i).wait()
        pltpu.make_async_copy(ohc_hbm.at[pl.ds(0, CHUNK)], ohbuf,
                              sem_i).wait()
        coff = c * N_NODES

        @pl.loop(0, CHUNK // 16)
        def _ids(t):
            sl = pl.ds(16 * t, 16)
            dv = dstraw[sl]
            srcidx[sl] = srcraw[sl] + coff
            dgidx[sl] = jnp.minimum(dv, N_NODES - 1) + coff
            lv = dv - p * PH_NODES
            ok = (lv >= 0) & (lv < PH_NODES)
            spread = dv & 63
            dsbuf[sl] = jnp.where(ok, lv, DUMMY_N + spread)
            dswbuf[sl] = jnp.where(ok, lv >> 3, DUMMY_W + (dv & 7))

        pltpu.async_copy(xl_hbm.at[srcidx], xlrows, sem_g)
        pltpu.async_copy(xr_hbm.at[dgidx], xrrows, sem_g)

    def drain_scatters():
        pltpu.make_async_copy(xl_hbm.at[pl.ds(0, CHUNK)], stg, sem_s0).wait()
        pltpu.make_async_copy(xr_hbm.at[pl.ds(0, CHUNK)], stgw,
                              sem_s1).wait()

    def consume(S):
        (_, _, ohbuf, _, _, dsbuf, dswbuf, xlrows, xrrows, _, sem_g) = S
        pltpu.make_async_copy(xl_hbm.at[pl.ds(0, CHUNK)], xlrows,
                              sem_g).wait()
        pltpu.make_async_copy(xr_hbm.at[pl.ds(0, CHUNK)], xrrows,
                              sem_g).wait()

        @pl.loop(0, CHUNK)
        def _edge(e):
            xs = [xlrows[e, pl.ds(16 * j, 16)] for j in range(8)]
            ws = []
            for q in range(2):
                tsum = None
                for j in range(4 * q, 4 * q + 4):
                    z = xs[j] + xrrows[e, pl.ds(16 * j, 16)]
                    t = jnp.maximum(z, z * 0.2) * attbuf[pl.ds(16 * j, 16)]
                    tsum = t if tsum is None else tsum + t
                a = jnp.sum(tsum)
                ws.append(jnp.exp(jnp.full((16,), a, jnp.float32)))
            for j in range(8):
                stg[e, pl.ds(16 * j, 16)] = xs[j] * ws[j // 4]
            oc = ohbuf[e, pl.ds(0, 16)]
            zero = jnp.zeros((16,), jnp.float32)
            stgw[e, pl.ds(0, 16)] = (jnp.where(oc == 1.0, ws[0], zero) +
                                     jnp.where(oc == 2.0, ws[1], zero))

        pltpu.async_copy(stg, accn.at[dsbuf], sem_s0, add=True)
        pltpu.async_copy(stgw, accw.at[dswbuf], sem_s1, add=True)

    pltpu.sync_copy(att_hbm.at[pl.ds(c * HALF, HALF)], attbuf)

    for p in range(2):
        @pl.loop(0, CHUNK)
        def _zs(i):
            for j in range(8):
                stg[i, pl.ds(16 * j, 16)] = jnp.zeros((16,), jnp.float32)
                stgw[i, pl.ds(16 * j, 16)] = jnp.zeros((16,), jnp.float32)

        for k in range(6):
            zid = s + N_TILES * k

            @pl.when(zid < NZCH)
            def _zn():
                pltpu.sync_copy(stg, accn.at[pl.ds(zid * CHUNK, CHUNK)])

        @pl.when(s < NZW)
        def _zw():
            pltpu.sync_copy(stgw, accw.at[pl.ds(s * CHUNK, CHUNK)])

        plsc.subcore_barrier()

        fetch(A, 0)
        stage(A, p)

        @pl.loop(0, NB2)
        def _body(t):
            fetch(B, 2 * t + 1)

            @pl.when(t > 0)
            def _d0():
                drain_scatters()

            consume(A)
            stage(B, p)

            @pl.when(t < NB2 - 1)
            def _f2():
                fetch(A, 2 * t + 2)

            drain_scatters()
            consume(B)

            @pl.when(t < NB2 - 1)
            def _s2():
                stage(A, p)

        drain_scatters()
        plsc.subcore_barrier()

        for k in range(8):
            oid = s + N_TILES * k

            @pl.when(oid < NOCH)
            def _on():
                pltpu.sync_copy(
                    accn.at[pl.ds(oid * OCH, OCH)],
                    outn_hbm.at[pl.ds(c * N_NODES + p * PH_NODES + oid * OCH,
                                      OCH)])

        @pl.when(s < WCP)
        def _ow():
            pltpu.sync_copy(
                accw.at[pl.ds(s * CHUNK, CHUNK)],
                outw_hbm.at[pl.ds(((2 * c + p) * WCP + s) * CHUNK, CHUNK)])

        plsc.subcore_barrier()


_sc_cp = pltpu.CompilerParams()
if "needs_layout_passes" in pltpu.CompilerParams.__dataclass_fields__:
    _sc_cp = dataclasses.replace(_sc_cp, needs_layout_passes=False)

_IDXB = [
    pltpu.VMEM((CHUNK,), jnp.int32),
    pltpu.VMEM((CHUNK,), jnp.int32),
    pltpu.VMEM((CHUNK, 16), jnp.float32),
    pltpu.VMEM((CHUNK,), jnp.int32),
    pltpu.VMEM((CHUNK,), jnp.int32),
    pltpu.VMEM((CHUNK,), jnp.int32),
    pltpu.VMEM((CHUNK,), jnp.int32),
    pltpu.VMEM((CHUNK, HALF), jnp.float32),
    pltpu.VMEM((CHUNK, HALF), jnp.float32),
]

_sc_edge = functools.partial(
    pl.kernel,
    compiler_params=_sc_cp,
    out_type=(
        jax.ShapeDtypeStruct((2 * N_NODES, HALF), jnp.float32),
        jax.ShapeDtypeStruct((4 * ACC_W_ROWS, HALF), jnp.float32),
    ),
    mesh=plsc.VectorSubcoreMesh(core_axis_name="c", subcore_axis_name="s"),
    scratch_types=_IDXB + _IDXB + [
        pltpu.VMEM((HALF,), jnp.float32),
        pltpu.VMEM((CHUNK, HALF), jnp.float32),
        pltpu.VMEM((CHUNK, HALF), jnp.float32),
        pltpu.VMEM_SHARED((ACC_N_ROWS, HALF), jnp.float32),
        pltpu.VMEM_SHARED((ACC_W_ROWS, HALF), jnp.float32),
        pltpu.SemaphoreType.DMA,
        pltpu.SemaphoreType.DMA,
        pltpu.SemaphoreType.DMA,
        pltpu.SemaphoreType.DMA,
        pltpu.SemaphoreType.DMA,
        pltpu.SemaphoreType.DMA,
    ],
)(_sc_edge_body)


def kernel(x, edge_index, W_l, b_l, W_r, b_r, att, bias, gamma, beta):
    ei = edge_index.astype(jnp.int32)
    loops = jnp.arange(N_NODES, dtype=jnp.int32)
    src = jnp.concatenate([ei[0], loops])
    dst = jnp.concatenate([ei[1], loops])
    pad = E_PAD - E_TOT
    srcp = jnp.concatenate([src, jnp.zeros((pad,), jnp.int32)])
    dstp = jnp.concatenate([dst, jnp.full((pad,), 2 * N_NODES, jnp.int32)])
    lane0 = 2 * (dstp & 7)
    lanes = jnp.arange(16, dtype=jnp.int32)
    ohc = ((lanes[None, :] == lane0[:, None]).astype(jnp.float32) +
           2.0 * (lanes[None, :] == (lane0 + 1)[:, None]).astype(jnp.float32))
    att1 = att.reshape(HC)

    xl = _mm(x, W_l, b_l)
    xr = _mm(x, W_r, b_r)
    xlt = xl.reshape(N_NODES, 2, HALF).transpose(1, 0, 2).reshape(-1, HALF)
    xrt = xr.reshape(N_NODES, 2, HALF).transpose(1, 0, 2).reshape(-1, HALF)

    outn, outw = _sc_edge(xlt, xrt, srcp, dstp, ohc, att1)
    halves = outn.reshape(2, N_NODES, HALF)
    wq = outw.reshape(2, 2, ACC_W_ROWS, HALF)[:, :, :, :16]
    wq = wq.reshape(2, 2, ACC_W_ROWS * 8, 2)[:, :, :PH_NODES, :]
    wq = wq.reshape(2, N_NODES, 2)
    w4 = jnp.concatenate([wq[0], wq[1]], axis=-1)

    return _ln_elu(halves, w4, bias, gamma, beta)


# speedup vs baseline: 17.0924x; 1.2651x over previous
import dataclasses
import functools

import jax
import jax.numpy as jnp
from jax import lax
from jax.experimental import pallas as pl
from jax.experimental.pallas import tpu as pltpu
from jax.experimental.pallas import tpu_sc as plsc

N_NODES = 10000
IN_CH = 256
HC = 256
HALF = 128
E_RAW = 160000
E_TOT = E_RAW + N_NODES

N_TILES = 16
CHUNK = 64
CH_PER_TILE = 168
NB2 = CH_PER_TILE // 2
E_PAD = N_TILES * CH_PER_TILE * CHUNK

PH_NODES = N_NODES // 2
ACC_N_ROWS = 5184
DUMMY_N = 5120
ACC_W_ROWS = 640
DUMMY_W = 630
NZCH = ACC_N_ROWS // CHUNK
NZW = ACC_W_ROWS // CHUNK
OCH = 40
NOCH = PH_NODES // OCH
WCP = ACC_W_ROWS // CHUNK

_ROWS = 400


def _mm_body(x_ref, w_ref, b_ref, o_ref):
    o_ref[...] = jnp.dot(x_ref[...], w_ref[...],
                         preferred_element_type=jnp.float32) + b_ref[0]


def _mm(x, W, b):
    n_blk = N_NODES // _ROWS
    return pl.pallas_call(
        _mm_body,
        grid=(n_blk,),
        in_specs=[
            pl.BlockSpec((_ROWS, IN_CH), lambda i: (i, 0)),
            pl.BlockSpec((IN_CH, HC), lambda i: (0, 0)),
            pl.BlockSpec((1, HC), lambda i: (0, 0)),
        ],
        out_specs=pl.BlockSpec((_ROWS, HC), lambda i: (i, 0)),
        out_shape=jax.ShapeDtypeStruct((N_NODES, HC), jnp.float32),
    )(x, W, b.reshape(1, HC))


def _ln_elu_body(h_ref, w_ref, bias_ref, gamma_ref, beta_ref, o_ref):
    num = jnp.concatenate([h_ref[0], h_ref[1]], axis=-1)
    pieces = []
    for h in range(4):
        inv = 1.0 / (w_ref[:, h:h + 1] + 1e-16)
        pieces.append(num[:, 64 * h:64 * (h + 1)] * inv)
    full = jnp.concatenate(pieces, axis=-1) + bias_ref[0]
    mean = jnp.mean(full, axis=-1, keepdims=True)
    var = jnp.mean((full - mean) ** 2, axis=-1, keepdims=True)
    y = (full - mean) / jnp.sqrt(var + 1e-5) * gamma_ref[0] + beta_ref[0]
    o_ref[...] = jnp.where(y > 0, y, jnp.exp(jnp.minimum(y, 0.0)) - 1.0)


def _ln_elu(halves, w4, bias, gamma, beta):
    n_blk = N_NODES // _ROWS
    return pl.pallas_call(
        _ln_elu_body,
        grid=(n_blk,),
        in_specs=[
            pl.BlockSpec((2, _ROWS, HALF), lambda i: (0, i, 0)),
            pl.BlockSpec((_ROWS, 4), lambda i: (i, 0)),
            pl.BlockSpec((1, HC), lambda i: (0, 0)),
            pl.BlockSpec((1, HC), lambda i: (0, 0)),
            pl.BlockSpec((1, HC), lambda i: (0, 0)),
        ],
        out_specs=pl.BlockSpec((_ROWS, HC), lambda i: (i, 0)),
        out_shape=jax.ShapeDtypeStruct((N_NODES, HC), jnp.float32),
    )(halves, w4, bias.reshape(1, HC), gamma.reshape(1, HC),
      beta.reshape(1, HC))


def _sc_edge_body(xl_hbm, xr_hbm, src_hbm, dst_hbm, ohc_hbm, att_hbm,
                  outn_hbm, outw_hbm,
                  srcrawA, dstrawA, ohbufA, srcidxA, dgidxA, dsbufA, dswbufA,
                  xlrowsA, xrrowsA,
                  srcrawB, dstrawB, ohbufB, srcidxB, dgidxB, dsbufB, dswbufB,
                  xlrowsB, xrrowsB,
                  attbuf, stg, stgw, accn, accw,
                  sem_iA, sem_iB, sem_gA, sem_gB, sem_s0, sem_s1):
    c = lax.axis_index("c")
    s = lax.axis_index("s")

    A = (srcrawA, dstrawA, ohbufA, srcidxA, dgidxA, dsbufA, dswbufA,
         xlrowsA, xrrowsA, sem_iA, sem_gA)
    B = (srcrawB, dstrawB, ohbufB, srcidxB, dgidxB, dsbufB, dswbufB,
         xlrowsB, xrrowsB, sem_iB, sem_gB)

    def fetch(S, chunk):
        srcraw, dstraw, ohbuf, _, _, _, _, _, _, sem_i, _ = S
        base = (s * CH_PER_TILE + chunk) * CHUNK
        pltpu.async_copy(src_hbm.at[pl.ds(base, CHUNK)], srcraw, sem_i)
        pltpu.async_copy(dst_hbm.at[pl.ds(base, CHUNK)], dstraw, sem_i)
        pltpu.async_copy(ohc_hbm.at[pl.ds(base, CHUNK)], ohbuf, sem_i)

    def stage(S, p):
        (srcraw, dstraw, ohbuf, srcidx, dgidx, dsbuf, dswbuf,
         xlrows, xrrows, sem_i, sem_g) = S
        pltpu.make_async_copy(src_hbm.at[pl.ds(0, CHUNK)], srcraw,
                              sem_i).wait()
        pltpu.make_async_copy(dst_hbm.at[pl.ds(0, CHUNK)], dstraw,
                              sem_i).wait()
        pltpu.make_async_copy(ohc_hbm.at[pl.ds(0, CHUNK)], ohbuf,
                              sem_i).wait()
        coff = c * N_NODES

        @plsc.parallel_loop(0, CHUNK // 16)
        def _ids(t):
            sl = pl.ds(16 * t, 16)
            dv = dstraw[sl]
            srcidx[sl] = srcraw[sl] + coff
            dgidx[sl] = jnp.minimum(dv, N_NODES - 1) + coff
            lv = dv - p * PH_NODES
            ok = (lv >= 0) & (lv < PH_NODES)
            spread = dv & 63
            dsbuf[sl] = jnp.where(ok, lv, DUMMY_N + spread)
            dswbuf[sl] = jnp.where(ok, lv >> 3, DUMMY_W + (dv & 7))

        pltpu.async_copy(xl_hbm.at[srcidx], xlrows, sem_g)
        pltpu.async_copy(xr_hbm.at[dgidx], xrrows, sem_g)

    def drain_scatters():
        pltpu.make_async_copy(xl_hbm.at[pl.ds(0, CHUNK)], stg, sem_s0).wait()
        pltpu.make_async_copy(xr_hbm.at[pl.ds(0, CHUNK)], stgw,
                              sem_s1).wait()

    def consume(S):
        (_, _, ohbuf, _, _, dsbuf, dswbuf, xlrows, xrrows, _, sem_g) = S
        pltpu.make_async_copy(xl_hbm.at[pl.ds(0, CHUNK)], xlrows,
                              sem_g).wait()
        pltpu.make_async_copy(xr_hbm.at[pl.ds(0, CHUNK)], xrrows,
                              sem_g).wait()

        @plsc.parallel_loop(0, CHUNK, unroll=2)
        def _edge(e):
            xs = [xlrows[e, pl.ds(16 * j, 16)] for j in range(8)]
            ws = []
            for q in range(2):
                tsum = None
                for j in range(4 * q, 4 * q + 4):
                    z = xs[j] + xrrows[e, pl.ds(16 * j, 16)]
                    t = jnp.maximum(z, z * 0.2) * attbuf[pl.ds(16 * j, 16)]
                    tsum = t if tsum is None else tsum + t
                a = jnp.sum(tsum)
                ws.append(jnp.exp(jnp.full((16,), a, jnp.float32)))
            for j in range(8):
                stg[e, pl.ds(16 * j, 16)] = xs[j] * ws[j // 4]
            oc = ohbuf[e, pl.ds(0, 16)]
            zero = jnp.zeros((16,), jnp.float32)
            stgw[e, pl.ds(0, 16)] = (jnp.where(oc == 1.0, ws[0], zero) +
                                     jnp.where(oc == 2.0, ws[1], zero))

        pltpu.async_copy(stg, accn.at[dsbuf], sem_s0, add=True)
        pltpu.async_copy(stgw, accw.at[dswbuf], sem_s1, add=True)

    pltpu.sync_copy(att_hbm.at[pl.ds(c * HALF, HALF)], attbuf)

    for p in range(2):
        @pl.loop(0, CHUNK)
        def _zs(i):
            for j in range(8):
                stg[i, pl.ds(16 * j, 16)] = jnp.zeros((16,), jnp.float32)
                stgw[i, pl.ds(16 * j, 16)] = jnp.zeros((16,), jnp.float32)

        for k in range(6):
            zid = s + N_TILES * k

            @pl.when(zid < NZCH)
            def _zn():
                pltpu.sync_copy(stg, accn.at[pl.ds(zid * CHUNK, CHUNK)])

        @pl.when(s < NZW)
        def _zw():
            pltpu.sync_copy(stgw, accw.at[pl.ds(s * CHUNK, CHUNK)])

        plsc.subcore_barrier()

        fetch(A, 0)
        stage(A, p)

        @pl.loop(0, NB2)
        def _body(t):
            fetch(B, 2 * t + 1)

            @pl.when(t > 0)
            def _d0():
                drain_scatters()

            consume(A)
            stage(B, p)

            @pl.when(t < NB2 - 1)
            def _f2():
                fetch(A, 2 * t + 2)

            drain_scatters()
            consume(B)

            @pl.when(t < NB2 - 1)
            def _s2():
                stage(A, p)

        drain_scatters()
        plsc.subcore_barrier()

        for k in range(8):
            oid = s + N_TILES * k

            @pl.when(oid < NOCH)
            def _on():
                pltpu.sync_copy(
                    accn.at[pl.ds(oid * OCH, OCH)],
                    outn_hbm.at[pl.ds(c * N_NODES + p * PH_NODES + oid * OCH,
                                      OCH)])

        @pl.when(s < WCP)
        def _ow():
            pltpu.sync_copy(
                accw.at[pl.ds(s * CHUNK, CHUNK)],
                outw_hbm.at[pl.ds(((2 * c + p) * WCP + s) * CHUNK, CHUNK)])

        plsc.subcore_barrier()


_sc_cp = pltpu.CompilerParams()
if "needs_layout_passes" in pltpu.CompilerParams.__dataclass_fields__:
    _sc_cp = dataclasses.replace(_sc_cp, needs_layout_passes=False)

_IDXB = [
    pltpu.VMEM((CHUNK,), jnp.int32),
    pltpu.VMEM((CHUNK,), jnp.int32),
    pltpu.VMEM((CHUNK, 16), jnp.float32),
    pltpu.VMEM((CHUNK,), jnp.int32),
    pltpu.VMEM((CHUNK,), jnp.int32),
    pltpu.VMEM((CHUNK,), jnp.int32),
    pltpu.VMEM((CHUNK,), jnp.int32),
    pltpu.VMEM((CHUNK, HALF), jnp.float32),
    pltpu.VMEM((CHUNK, HALF), jnp.float32),
]

_sc_edge = functools.partial(
    pl.kernel,
    compiler_params=_sc_cp,
    out_type=(
        jax.ShapeDtypeStruct((2 * N_NODES, HALF), jnp.float32),
        jax.ShapeDtypeStruct((4 * ACC_W_ROWS, HALF), jnp.float32),
    ),
    mesh=plsc.VectorSubcoreMesh(core_axis_name="c", subcore_axis_name="s"),
    scratch_types=_IDXB + _IDXB + [
        pltpu.VMEM((HALF,), jnp.float32),
        pltpu.VMEM((CHUNK, HALF), jnp.float32),
        pltpu.VMEM((CHUNK, HALF), jnp.float32),
        pltpu.VMEM_SHARED((ACC_N_ROWS, HALF), jnp.float32),
        pltpu.VMEM_SHARED((ACC_W_ROWS, HALF), jnp.float32),
        pltpu.SemaphoreType.DMA,
        pltpu.SemaphoreType.DMA,
        pltpu.SemaphoreType.DMA,
        pltpu.SemaphoreType.DMA,
        pltpu.SemaphoreType.DMA,
        pltpu.SemaphoreType.DMA,
    ],
)(_sc_edge_body)


def kernel(x, edge_index, W_l, b_l, W_r, b_r, att, bias, gamma, beta):
    ei = edge_index.astype(jnp.int32)
    loops = jnp.arange(N_NODES, dtype=jnp.int32)
    src = jnp.concatenate([ei[0], loops])
    dst = jnp.concatenate([ei[1], loops])
    pad = E_PAD - E_TOT
    srcp = jnp.concatenate([src, jnp.zeros((pad,), jnp.int32)])
    dstp = jnp.concatenate([dst, jnp.full((pad,), 2 * N_NODES, jnp.int32)])
    lane0 = 2 * (dstp & 7)
    lanes = jnp.arange(16, dtype=jnp.int32)
    ohc = ((lanes[None, :] == lane0[:, None]).astype(jnp.float32) +
           2.0 * (lanes[None, :] == (lane0 + 1)[:, None]).astype(jnp.float32))
    att1 = att.reshape(HC)

    xl = _mm(x, W_l, b_l)
    xr = _mm(x, W_r, b_r)
    xlt = xl.reshape(N_NODES, 2, HALF).transpose(1, 0, 2).reshape(-1, HALF)
    xrt = xr.reshape(N_NODES, 2, HALF).transpose(1, 0, 2).reshape(-1, HALF)

    outn, outw = _sc_edge(xlt, xrt, srcp, dstp, ohc, att1)
    halves = outn.reshape(2, N_NODES, HALF)
    wq = outw.reshape(2, 2, ACC_W_ROWS, HALF)[:, :, :, :16]
    wq = wq.reshape(2, 2, ACC_W_ROWS * 8, 2)[:, :, :PH_NODES, :]
    wq = wq.reshape(2, N_NODES, 2)
    w4 = jnp.concatenate([wq[0], wq[1]], axis=-1)

    return _ln_elu(halves, w4, bias, gamma, beta)
